# baseline (device time: 490999 ns/iter reference)
import jax
import jax.numpy as jnp
from jax import lax
from jax.experimental import pallas as pl
from jax.experimental.pallas import tpu as pltpu

NZ = 4


def kernel(x, dest):
    m, n = x.shape
    d2 = dest.reshape(16, 128)

    def body(x_ref, d_ref, ox_ref, od_ref, sxs, sxr, sds, sdr):
        my_x = lax.axis_index("x")
        my_y = lax.axis_index("y")
        my_z = lax.axis_index("z")
        left = (my_z - 1) % NZ
        right = (my_z + 1) % NZ

        barrier = pltpu.get_barrier_semaphore()
        for nz in (left, right):
            pl.semaphore_signal(
                barrier, inc=1,
                device_id=(my_x, my_y, nz),
                device_id_type=pl.DeviceIdType.MESH,
            )
        pl.semaphore_wait(barrier, 2)

        ox_ref[my_z] = x_ref[:, :]
        od_ref[my_z] = d_ref[:, :]

        for h in range(NZ - 1):
            origin = (my_z - h) % NZ
            rx = pltpu.make_async_remote_copy(
                src_ref=ox_ref.at[origin],
                dst_ref=ox_ref.at[origin],
                send_sem=sxs.at[h],
                recv_sem=sxr.at[h],
                device_id=(my_x, my_y, right),
                device_id_type=pl.DeviceIdType.MESH,
            )
            rd = pltpu.make_async_remote_copy(
                src_ref=od_ref.at[origin],
                dst_ref=od_ref.at[origin],
                send_sem=sds.at[h],
                recv_sem=sdr.at[h],
                device_id=(my_x, my_y, right),
                device_id_type=pl.DeviceIdType.MESH,
            )
            rx.start()
            rd.start()
            rx.wait()
            rd.wait()

    xg, dg = pl.pallas_call(
        body,
        out_shape=[
            jax.ShapeDtypeStruct((NZ, m, n), x.dtype),
            jax.ShapeDtypeStruct((NZ, 16, 128), d2.dtype),
        ],
        in_specs=[
            pl.BlockSpec(memory_space=pltpu.VMEM),
            pl.BlockSpec(memory_space=pltpu.VMEM),
        ],
        out_specs=[
            pl.BlockSpec(memory_space=pltpu.VMEM),
            pl.BlockSpec(memory_space=pltpu.VMEM),
        ],
        scratch_shapes=[
            pltpu.SemaphoreType.DMA((NZ - 1,)),
            pltpu.SemaphoreType.DMA((NZ - 1,)),
            pltpu.SemaphoreType.DMA((NZ - 1,)),
            pltpu.SemaphoreType.DMA((NZ - 1,)),
        ],
        compiler_params=pltpu.CompilerParams(collective_id=0),
    )(x, d2)

    my_z = lax.axis_index("z")
    dg_flat = dg.reshape(NZ * m)
    idx = jnp.nonzero(dg_flat == my_z, size=m, fill_value=0)[0]
    return xg.reshape(NZ * m, n)[idx]


# device time: 112217 ns/iter; 4.3754x vs baseline; 4.3754x over previous
import jax
import jax.numpy as jnp
from jax import lax
from jax.experimental import pallas as pl
from jax.experimental.pallas import tpu as pltpu

NZ = 4


def kernel(x, dest):
    m, n = x.shape

    onehot = (dest[:, None] == jnp.arange(NZ, dtype=dest.dtype)).astype(jnp.int32)
    lrank = jnp.take_along_axis(
        jnp.cumsum(onehot, axis=0) - onehot, dest[:, None].astype(jnp.int32), axis=1
    )[:, 0]
    counts = jnp.sum(onehot, axis=0)
    counts_v = jnp.zeros((8, 128), jnp.int32).at[0, :NZ].set(counts)

    def body(
        x_ref,
        dest_ref,
        lrank_ref,
        counts_ref,
        cv_ref,
        out_ref,
        cg_ref,
        csm_ref,
        base_ref,
        csend_sems,
        crecv_sems,
        cloc_sem,
        ssem,
        rsem,
    ):
        my_x = lax.axis_index("x")
        my_y = lax.axis_index("y")
        my_z = lax.axis_index("z")

        barrier = pltpu.get_barrier_semaphore()
        for dz in range(1, NZ):
            pl.semaphore_signal(
                barrier, inc=1,
                device_id=(my_x, my_y, (my_z + dz) % NZ),
                device_id_type=pl.DeviceIdType.MESH,
            )
        pl.semaphore_wait(barrier, NZ - 1)

        cg_ref[my_z] = cv_ref[:, :]
        for i, dz in enumerate(range(1, NZ)):
            peer = (my_z + dz) % NZ
            c_rdma = pltpu.make_async_remote_copy(
                src_ref=cv_ref,
                dst_ref=cg_ref.at[my_z],
                send_sem=csend_sems.at[i],
                recv_sem=crecv_sems.at[my_z],
                device_id=(my_x, my_y, peer),
                device_id_type=pl.DeviceIdType.MESH,
            )
            c_rdma.start()
        for i, dz in enumerate(range(1, NZ)):
            src_z = (my_z - dz) % NZ
            c_wait = pltpu.make_async_remote_copy(
                src_ref=cv_ref,
                dst_ref=cg_ref.at[src_z],
                send_sem=csend_sems.at[i],
                recv_sem=crecv_sems.at[src_z],
                device_id=(my_x, my_y, src_z),
                device_id_type=pl.DeviceIdType.MESH,
            )
            c_wait.wait_send()
            c_wait.wait_recv()

        c_cp = pltpu.make_async_copy(cg_ref, csm_ref, cloc_sem)
        c_cp.start()
        c_cp.wait()

        for k in range(NZ):
            acc = jnp.int32(0)
            for zp in range(NZ):
                acc = acc + jnp.where(zp < my_z, csm_ref[zp, 0, k], 0)
            base_ref[k] = acc

        def row_body(r, _):
            d = dest_ref[r]
            off = base_ref[d] + lrank_ref[r]

            @pl.when(d != my_z)
            def _remote():
                rdma = pltpu.make_async_remote_copy(
                    src_ref=x_ref.at[pl.ds(r, 1)],
                    dst_ref=out_ref.at[pl.ds(off, 1)],
                    send_sem=ssem,
                    recv_sem=rsem,
                    device_id=(my_x, my_y, d),
                    device_id_type=pl.DeviceIdType.MESH,
                )
                rdma.start()

            @pl.when(d == my_z)
            def _local():
                out_ref[pl.ds(off, 1), :] = x_ref[pl.ds(r, 1), :]

            return _

        lax.fori_loop(0, m, row_body, None)

        n_io = m - counts_ref[my_z]

        def drain(i, _):
            dummy = pltpu.make_async_remote_copy(
                src_ref=x_ref.at[pl.ds(0, 1)],
                dst_ref=out_ref.at[pl.ds(0, 1)],
                send_sem=ssem,
                recv_sem=rsem,
                device_id=(my_x, my_y, (my_z + 1) % NZ),
                device_id_type=pl.DeviceIdType.MESH,
            )
            dummy.wait_send()
            dummy.wait_recv()
            return _

        lax.fori_loop(0, n_io, drain, None)

    return pl.pallas_call(
        body,
        out_shape=jax.ShapeDtypeStruct((m, n), x.dtype),
        in_specs=[
            pl.BlockSpec(memory_space=pltpu.VMEM),
            pl.BlockSpec(memory_space=pltpu.SMEM),
            pl.BlockSpec(memory_space=pltpu.SMEM),
            pl.BlockSpec(memory_space=pltpu.SMEM),
            pl.BlockSpec(memory_space=pltpu.VMEM),
        ],
        out_specs=pl.BlockSpec(memory_space=pltpu.VMEM),
        scratch_shapes=[
            pltpu.VMEM((NZ, 8, 128), jnp.int32),
            pltpu.SMEM((NZ, 8, 128), jnp.int32),
            pltpu.SMEM((NZ,), jnp.int32),
            pltpu.SemaphoreType.DMA((NZ - 1,)),
            pltpu.SemaphoreType.DMA((NZ,)),
            pltpu.SemaphoreType.DMA,
            pltpu.SemaphoreType.DMA,
            pltpu.SemaphoreType.DMA,
        ],
        compiler_params=pltpu.CompilerParams(collective_id=0),
    )(x, dest.astype(jnp.int32), lrank, counts, counts_v)


# device time: 107127 ns/iter; 4.5833x vs baseline; 1.0475x over previous
import jax
import jax.numpy as jnp
from jax import lax
from jax.experimental import pallas as pl
from jax.experimental.pallas import tpu as pltpu

NZ = 4


def kernel(x, dest):
    m, n = x.shape

    dest = dest.astype(jnp.int32)
    counts = jnp.sum(
        (dest[:, None] == jnp.arange(NZ, dtype=jnp.int32)).astype(jnp.int32), axis=0
    )

    def body(
        x_ref,
        dest_ref,
        counts_ref,
        out_ref,
        cg_ref,
        csm_ref,
        base_ref,
        cnt_ref,
        csend_sems,
        crecv_sems,
        cloc_sem,
        ssem,
        rsem,
    ):
        my_x = lax.axis_index("x")
        my_y = lax.axis_index("y")
        my_z = lax.axis_index("z")

        barrier = pltpu.get_barrier_semaphore()
        for dz in range(1, NZ):
            pl.semaphore_signal(
                barrier, inc=1,
                device_id=(my_x, my_y, (my_z + dz) % NZ),
                device_id_type=pl.DeviceIdType.MESH,
            )
        pl.semaphore_wait(barrier, NZ - 1)

        row_i = lax.broadcasted_iota(jnp.int32, (8, 128), 0)
        lane_i = lax.broadcasted_iota(jnp.int32, (8, 128), 1)
        cvv = jnp.zeros((8, 128), jnp.int32)
        for k in range(NZ):
            cvv = jnp.where((row_i == 0) & (lane_i == k), counts_ref[k], cvv)
        cg_ref[my_z] = cvv

        for i, dz in enumerate(range(1, NZ)):
            peer = (my_z + dz) % NZ
            c_rdma = pltpu.make_async_remote_copy(
                src_ref=cg_ref.at[my_z],
                dst_ref=cg_ref.at[my_z],
                send_sem=csend_sems.at[i],
                recv_sem=crecv_sems.at[my_z],
                device_id=(my_x, my_y, peer),
                device_id_type=pl.DeviceIdType.MESH,
            )
            c_rdma.start()
        for i, dz in enumerate(range(1, NZ)):
            src_z = (my_z - dz) % NZ
            c_wait = pltpu.make_async_remote_copy(
                src_ref=cg_ref.at[my_z],
                dst_ref=cg_ref.at[src_z],
                send_sem=csend_sems.at[i],
                recv_sem=crecv_sems.at[src_z],
                device_id=(my_x, my_y, src_z),
                device_id_type=pl.DeviceIdType.MESH,
            )
            c_wait.wait_send()
            c_wait.wait_recv()

        c_cp = pltpu.make_async_copy(cg_ref, csm_ref, cloc_sem)
        c_cp.start()
        c_cp.wait()

        for k in range(NZ):
            acc = jnp.int32(0)
            for zp in range(NZ):
                acc = acc + jnp.where(zp < my_z, csm_ref[zp, 0, k], 0)
            base_ref[k] = acc
            cnt_ref[k] = 0

        def row_body(r, _):
            d = dest_ref[r]
            c = cnt_ref[d]
            off = base_ref[d] + c
            cnt_ref[d] = c + 1

            @pl.when(d != my_z)
            def _remote():
                rdma = pltpu.make_async_remote_copy(
                    src_ref=x_ref.at[pl.ds(r, 1)],
                    dst_ref=out_ref.at[pl.ds(off, 1)],
                    send_sem=ssem,
                    recv_sem=rsem,
                    device_id=(my_x, my_y, d),
                    device_id_type=pl.DeviceIdType.MESH,
                )
                rdma.start()

            @pl.when(d == my_z)
            def _local():
                out_ref[pl.ds(off, 1), :] = x_ref[pl.ds(r, 1), :]

            return _

        lax.fori_loop(0, m, row_body, None)

        n_io = m - counts_ref[my_z]

        def drain(i, _):
            dummy = pltpu.make_async_remote_copy(
                src_ref=x_ref.at[pl.ds(0, 1)],
                dst_ref=out_ref.at[pl.ds(0, 1)],
                send_sem=ssem,
                recv_sem=rsem,
                device_id=(my_x, my_y, (my_z + 1) % NZ),
                device_id_type=pl.DeviceIdType.MESH,
            )
            dummy.wait_send()
            dummy.wait_recv()
            return _

        lax.fori_loop(0, n_io, drain, None)

    return pl.pallas_call(
        body,
        out_shape=jax.ShapeDtypeStruct((m, n), x.dtype),
        in_specs=[
            pl.BlockSpec(memory_space=pltpu.VMEM),
            pl.BlockSpec(memory_space=pltpu.SMEM),
            pl.BlockSpec(memory_space=pltpu.SMEM),
        ],
        out_specs=pl.BlockSpec(memory_space=pltpu.VMEM),
        scratch_shapes=[
            pltpu.VMEM((NZ, 8, 128), jnp.int32),
            pltpu.SMEM((NZ, 8, 128), jnp.int32),
            pltpu.SMEM((NZ,), jnp.int32),
            pltpu.SMEM((NZ,), jnp.int32),
            pltpu.SemaphoreType.DMA((NZ - 1,)),
            pltpu.SemaphoreType.DMA((NZ,)),
            pltpu.SemaphoreType.DMA,
            pltpu.SemaphoreType.DMA,
            pltpu.SemaphoreType.DMA,
        ],
        compiler_params=pltpu.CompilerParams(collective_id=0),
    )(x, dest, counts)


# device time: 106975 ns/iter; 4.5898x vs baseline; 1.0014x over previous
import jax
import jax.numpy as jnp
from jax import lax
from jax.experimental import pallas as pl
from jax.experimental.pallas import tpu as pltpu

NZ = 4


def kernel(x, dest):
    m, n = x.shape

    dest = dest.astype(jnp.int32)
    counts = jnp.sum(
        (dest[:, None] == jnp.arange(NZ, dtype=jnp.int32)).astype(jnp.int32), axis=0
    )

    def body(
        x_ref,
        dest_ref,
        counts_ref,
        out_ref,
        cg_ref,
        csm_ref,
        base_ref,
        cnt_ref,
        csend_sems,
        crecv_sems,
        cloc_sem,
        ssem,
        rsem,
    ):
        my_x = lax.axis_index("x")
        my_y = lax.axis_index("y")
        my_z = lax.axis_index("z")

        barrier = pltpu.get_barrier_semaphore()
        for dz in range(1, NZ):
            pl.semaphore_signal(
                barrier, inc=1,
                device_id=(my_x, my_y, (my_z + dz) % NZ),
                device_id_type=pl.DeviceIdType.MESH,
            )
        pl.semaphore_wait(barrier, NZ - 1)

        row_i = lax.broadcasted_iota(jnp.int32, (8, 128), 0)
        lane_i = lax.broadcasted_iota(jnp.int32, (8, 128), 1)
        cvv = jnp.zeros((8, 128), jnp.int32)
        for k in range(NZ):
            cvv = jnp.where((row_i == 0) & (lane_i == k), counts_ref[k], cvv)
        cg_ref[my_z] = cvv

        for i, dz in enumerate(range(1, NZ)):
            peer = (my_z + dz) % NZ
            c_rdma = pltpu.make_async_remote_copy(
                src_ref=cg_ref.at[my_z],
                dst_ref=cg_ref.at[my_z],
                send_sem=csend_sems.at[i],
                recv_sem=crecv_sems.at[my_z],
                device_id=(my_x, my_y, peer),
                device_id_type=pl.DeviceIdType.MESH,
            )
            c_rdma.start()
        for i, dz in enumerate(range(1, NZ)):
            src_z = (my_z - dz) % NZ
            c_wait = pltpu.make_async_remote_copy(
                src_ref=cg_ref.at[my_z],
                dst_ref=cg_ref.at[src_z],
                send_sem=csend_sems.at[i],
                recv_sem=crecv_sems.at[src_z],
                device_id=(my_x, my_y, src_z),
                device_id_type=pl.DeviceIdType.MESH,
            )
            c_wait.wait_send()
            c_wait.wait_recv()

        c_cp = pltpu.make_async_copy(cg_ref, csm_ref, cloc_sem)
        c_cp.start()
        c_cp.wait()

        for k in range(NZ):
            acc = jnp.int32(0)
            for zp in range(NZ):
                acc = acc + jnp.where(zp < my_z, csm_ref[zp, 0, k], 0)
            base_ref[k] = acc
            cnt_ref[k] = 0

        def row_body(r, _):
            d = dest_ref[r]
            c = cnt_ref[d]
            off = base_ref[d] + c
            cnt_ref[d] = c + 1

            @pl.when(d != my_z)
            def _remote():
                rdma = pltpu.make_async_remote_copy(
                    src_ref=x_ref.at[pl.ds(r, 1)],
                    dst_ref=out_ref.at[pl.ds(off, 1)],
                    send_sem=ssem,
                    recv_sem=rsem,
                    device_id=(my_x, my_y, d),
                    device_id_type=pl.DeviceIdType.MESH,
                )
                rdma.start()

            return _

        lax.fori_loop(0, m, row_body, None)

        def local_body(r, c2):
            d = dest_ref[r]

            @pl.when(d == my_z)
            def _local():
                out_ref[pl.ds(base_ref[my_z] + c2, 1), :] = x_ref[pl.ds(r, 1), :]

            return c2 + jnp.where(d == my_z, 1, 0)

        lax.fori_loop(0, m, local_body, jnp.int32(0))

        n_io = m - counts_ref[my_z]

        def drain(i, _):
            dummy = pltpu.make_async_remote_copy(
                src_ref=x_ref.at[pl.ds(0, 1)],
                dst_ref=out_ref.at[pl.ds(0, 1)],
                send_sem=ssem,
                recv_sem=rsem,
                device_id=(my_x, my_y, (my_z + 1) % NZ),
                device_id_type=pl.DeviceIdType.MESH,
            )
            dummy.wait_send()
            dummy.wait_recv()
            return _

        lax.fori_loop(0, n_io, drain, None)

    return pl.pallas_call(
        body,
        out_shape=jax.ShapeDtypeStruct((m, n), x.dtype),
        in_specs=[
            pl.BlockSpec(memory_space=pltpu.VMEM),
            pl.BlockSpec(memory_space=pltpu.SMEM),
            pl.BlockSpec(memory_space=pltpu.SMEM),
        ],
        out_specs=pl.BlockSpec(memory_space=pltpu.VMEM),
        scratch_shapes=[
            pltpu.VMEM((NZ, 8, 128), jnp.int32),
            pltpu.SMEM((NZ, 8, 128), jnp.int32),
            pltpu.SMEM((NZ,), jnp.int32),
            pltpu.SMEM((NZ,), jnp.int32),
            pltpu.SemaphoreType.DMA((NZ - 1,)),
            pltpu.SemaphoreType.DMA((NZ,)),
            pltpu.SemaphoreType.DMA,
            pltpu.SemaphoreType.DMA,
            pltpu.SemaphoreType.DMA,
        ],
        compiler_params=pltpu.CompilerParams(collective_id=0),
    )(x, dest, counts)
